# baseline (speedup 1.0000x reference)
import jax
import jax.numpy as jnp
from jax import lax
from jax.experimental import pallas as pl
from jax.experimental.pallas import tpu as pltpu
from jax.experimental.pallas import tpu_sc as plsc

B, T, D = 16, 4096, 1024
NC, NS, L = 2, 16, 16
LG = D // L
TCH = 32
ROWS_W = T // 2
NCH = ROWS_W // TCH


def _tree(buf, col):
    vs = [buf[t, pl.ds(col, L)] for t in range(TCH)]
    while len(vs) > 1:
        nxt = [vs[i] + vs[i + 1] for i in range(0, len(vs) - 1, 2)]
        if len(vs) % 2:
            nxt.append(vs[-1])
        vs = nxt
    return vs[0]


def _body(x_hbm, lens_hbm, out_hbm, buf0, buf1, acc, lens_v, spmem, sem0, sem1):
    c = lax.axis_index("c")
    s = lax.axis_index("s")
    b = c * 8 + s // 2
    h = s % 2
    row0 = h * ROWS_W

    zeros = jnp.zeros((L,), jnp.float32)
    for j in range(LG):
        acc[pl.ds(j * L, L)] = zeros

    def src(chunk):
        return x_hbm.at[b, pl.ds(row0 + chunk * TCH, TCH), :]

    pltpu.make_async_copy(src(0), buf0, sem0).start()
    pltpu.make_async_copy(src(1), buf1, sem1).start()

    def accumulate(buf):
        @plsc.parallel_loop(0, LG, 1, unroll=8)
        def _(j):
            col = j * L
            acc[pl.ds(col, L)] = acc[pl.ds(col, L)] + _tree(buf, col)

    def pair(i, _):
        c0 = 2 * i
        pltpu.make_async_copy(src(c0), buf0, sem0).wait()
        accumulate(buf0)

        @pl.when(c0 + 2 < NCH)
        def _():
            pltpu.make_async_copy(src(c0 + 2), buf0, sem0).start()

        pltpu.make_async_copy(src(c0 + 1), buf1, sem1).wait()
        accumulate(buf1)

        @pl.when(c0 + 3 < NCH)
        def _():
            pltpu.make_async_copy(src(c0 + 3), buf1, sem1).start()

        return 0

    lax.fori_loop(0, NCH // 2, pair, 0)

    pltpu.sync_copy(lens_hbm, lens_v)
    lens_f = lens_v[...].astype(jnp.float32)
    lens_f = jnp.where(lens_f == 0.0, jnp.float32(1.5), lens_f)
    idx = jnp.full((L,), b, dtype=jnp.int32)
    dnums = lax.GatherDimensionNumbers(
        offset_dims=(), collapsed_slice_dims=(0,), start_index_map=(0,))
    my_len = lax.gather(lens_f, idx[:, None], dnums, slice_sizes=(1,),
                        mode=lax.GatherScatterMode.PROMISE_IN_BOUNDS)
    for j in range(LG):
        acc[pl.ds(j * L, L)] = acc[pl.ds(j * L, L)] / my_len

    pltpu.sync_copy(acc, spmem.at[s])
    plsc.subcore_barrier()

    @pl.when(h == 0)
    def _():
        pltpu.sync_copy(spmem.at[s + 1], buf0.at[0])
        for j in range(LG):
            acc[pl.ds(j * L, L)] = acc[pl.ds(j * L, L)] + buf0[0, pl.ds(j * L, L)]
        pltpu.sync_copy(acc, out_hbm.at[b])


def kernel(x, lens):
    mesh = plsc.VectorSubcoreMesh(core_axis_name="c", subcore_axis_name="s")
    return pl.kernel(
        _body,
        out_type=jax.ShapeDtypeStruct((B, D), jnp.float32),
        mesh=mesh,
        scratch_types=[
            pltpu.VMEM((TCH, D), jnp.float32),
            pltpu.VMEM((TCH, D), jnp.float32),
            pltpu.VMEM((D,), jnp.float32),
            pltpu.VMEM((L,), jnp.int32),
            pltpu.VMEM_SHARED((NS, D), jnp.float32),
            pltpu.SemaphoreType.DMA,
            pltpu.SemaphoreType.DMA,
        ],
    )(x, lens)


# speedup vs baseline: 1.9318x; 1.9318x over previous
import jax
import jax.numpy as jnp
from jax import lax
from jax.experimental import pallas as pl
from jax.experimental.pallas import tpu as pltpu
from jax.experimental.pallas import tpu_sc as plsc

B, T, D = 16, 4096, 1024
NC, NS, L = 2, 16, 16
LG = D // L
TCH = 16
TSC = 1024
ROWS_W = TSC // 2
NCH = ROWS_W // TCH


def _tree(buf, col):
    vs = [buf[t, pl.ds(col, L)] for t in range(TCH)]
    while len(vs) > 1:
        nxt = [vs[i] + vs[i + 1] for i in range(0, len(vs) - 1, 2)]
        if len(vs) % 2:
            nxt.append(vs[-1])
        vs = nxt
    return vs[0]


def _body(x_hbm, lens_hbm, out_hbm, buf0, buf1, acc, lens_v, spmem, sem0, sem1):
    c = lax.axis_index("c")
    s = lax.axis_index("s")
    b = c * 8 + s // 2
    h = s % 2
    row0 = h * ROWS_W

    zeros = jnp.zeros((L,), jnp.float32)
    for j in range(LG):
        acc[pl.ds(j * L, L)] = zeros

    def src(chunk):
        return x_hbm.at[b, pl.ds(row0 + chunk * TCH, TCH), :]

    pltpu.make_async_copy(src(0), buf0, sem0).start()
    pltpu.make_async_copy(src(1), buf1, sem1).start()

    def accumulate(buf):
        @plsc.parallel_loop(0, LG, 1, unroll=4)
        def _(j):
            col = j * L
            acc[pl.ds(col, L)] = acc[pl.ds(col, L)] + _tree(buf, col)

    def pair(i, _):
        c0 = 2 * i
        pltpu.make_async_copy(src(c0), buf0, sem0).wait()
        accumulate(buf0)

        @pl.when(c0 + 2 < NCH)
        def _():
            pltpu.make_async_copy(src(c0 + 2), buf0, sem0).start()

        pltpu.make_async_copy(src(c0 + 1), buf1, sem1).wait()
        accumulate(buf1)

        @pl.when(c0 + 3 < NCH)
        def _():
            pltpu.make_async_copy(src(c0 + 3), buf1, sem1).start()

        return 0

    lax.fori_loop(0, NCH // 2, pair, 0)

    pltpu.sync_copy(lens_hbm, lens_v)
    lens_f = lens_v[...].astype(jnp.float32)
    lens_f = jnp.where(lens_f == 0.0, jnp.float32(1.5), lens_f)
    idx = jnp.full((L,), b, dtype=jnp.int32)
    dnums = lax.GatherDimensionNumbers(
        offset_dims=(), collapsed_slice_dims=(0,), start_index_map=(0,))
    my_len = lax.gather(lens_f, idx[:, None], dnums, slice_sizes=(1,),
                        mode=lax.GatherScatterMode.PROMISE_IN_BOUNDS)
    for j in range(LG):
        acc[pl.ds(j * L, L)] = acc[pl.ds(j * L, L)] / my_len

    pltpu.sync_copy(acc, spmem.at[s])
    plsc.subcore_barrier()

    @pl.when(h == 0)
    def _():
        pltpu.sync_copy(spmem.at[s + 1], buf0.at[0])
        for j in range(LG):
            acc[pl.ds(j * L, L)] = acc[pl.ds(j * L, L)] + buf0[0, pl.ds(j * L, L)]
        pltpu.sync_copy(acc, out_hbm.at[b])


TR = 1024


def _tc_body(lens_sm, x_ref, o_ref):
    t = pl.program_id(1)

    @pl.when(t == 0)
    def _():
        o_ref[...] = jnp.zeros_like(o_ref)

    o_ref[...] += jnp.sum(x_ref[...], axis=1, keepdims=True)

    @pl.when(t == pl.num_programs(1) - 1)
    def _():
        b = pl.program_id(0)
        lf = lens_sm[b].astype(jnp.float32)
        lf = jnp.where(lf == 0.0, jnp.float32(1.5), lf)
        o_ref[...] = o_ref[...] / lf


def _tc_mean(x, lens):
    return pl.pallas_call(
        _tc_body,
        grid_spec=pltpu.PrefetchScalarGridSpec(
            num_scalar_prefetch=1,
            grid=(B, (T - TSC) // TR),
            in_specs=[pl.BlockSpec(
                (1, TR, D), lambda b, t, lens_s: (b, TSC // TR + t, 0))],
            out_specs=pl.BlockSpec((1, 1, D), lambda b, t, lens_s: (b, 0, 0)),
        ),
        out_shape=jax.ShapeDtypeStruct((B, 1, D), jnp.float32),
        compiler_params=pltpu.CompilerParams(
            dimension_semantics=("parallel", "arbitrary")),
    )(lens, x).reshape(B, D)


def kernel(x, lens):
    return _sc_mean(x, lens) + _tc_mean(x, lens)


def _sc_mean(x, lens):
    mesh = plsc.VectorSubcoreMesh(core_axis_name="c", subcore_axis_name="s")
    return pl.kernel(
        _body,
        out_type=jax.ShapeDtypeStruct((B, D), jnp.float32),
        mesh=mesh,
        scratch_types=[
            pltpu.VMEM((TCH, D), jnp.float32),
            pltpu.VMEM((TCH, D), jnp.float32),
            pltpu.VMEM((D,), jnp.float32),
            pltpu.VMEM((L,), jnp.int32),
            pltpu.VMEM_SHARED((NS, D), jnp.float32),
            pltpu.SemaphoreType.DMA,
            pltpu.SemaphoreType.DMA,
        ],
    )(x, lens)


# speedup vs baseline: 1.9644x; 1.0169x over previous
import jax
import jax.numpy as jnp
from jax import lax
from jax.experimental import pallas as pl
from jax.experimental.pallas import tpu as pltpu
from jax.experimental.pallas import tpu_sc as plsc

B, T, D = 16, 4096, 1024
NC, NS, L = 2, 16, 16
LG = D // L
TCH = 16
TTC = 2624
ROWS_W = (T - TTC) // 2
NCH = ROWS_W // TCH


def _tree(buf, col):
    vs = [buf[t, pl.ds(col, L)] for t in range(TCH)]
    while len(vs) > 1:
        nxt = [vs[i] + vs[i + 1] for i in range(0, len(vs) - 1, 2)]
        if len(vs) % 2:
            nxt.append(vs[-1])
        vs = nxt
    return vs[0]


def _body(x_hbm, lens_hbm, out_hbm, buf0, buf1, acc, lens_v, spmem, sem0, sem1):
    c = lax.axis_index("c")
    s = lax.axis_index("s")
    b = c * 8 + s // 2
    h = s % 2
    row0 = TTC + h * ROWS_W

    zeros = jnp.zeros((L,), jnp.float32)
    for j in range(LG):
        acc[pl.ds(j * L, L)] = zeros

    def src(chunk):
        return x_hbm.at[b, pl.ds(row0 + chunk * TCH, TCH), :]

    pltpu.make_async_copy(src(0), buf0, sem0).start()
    pltpu.make_async_copy(src(1), buf1, sem1).start()

    def accumulate(buf):
        @plsc.parallel_loop(0, LG, 1, unroll=4)
        def _(j):
            col = j * L
            acc[pl.ds(col, L)] = acc[pl.ds(col, L)] + _tree(buf, col)

    def pair(i, _):
        c0 = 2 * i
        pltpu.make_async_copy(src(c0), buf0, sem0).wait()
        accumulate(buf0)

        @pl.when(c0 + 2 < NCH)
        def _():
            pltpu.make_async_copy(src(c0 + 2), buf0, sem0).start()

        pltpu.make_async_copy(src(c0 + 1), buf1, sem1).wait()
        accumulate(buf1)

        @pl.when(c0 + 3 < NCH)
        def _():
            pltpu.make_async_copy(src(c0 + 3), buf1, sem1).start()

        return 0

    lax.fori_loop(0, NCH // 2, pair, 0)

    pltpu.sync_copy(lens_hbm, lens_v)
    lens_f = lens_v[...].astype(jnp.float32)
    lens_f = jnp.where(lens_f == 0.0, jnp.float32(1.5), lens_f)
    idx = jnp.full((L,), b, dtype=jnp.int32)
    dnums = lax.GatherDimensionNumbers(
        offset_dims=(), collapsed_slice_dims=(0,), start_index_map=(0,))
    my_len = lax.gather(lens_f, idx[:, None], dnums, slice_sizes=(1,),
                        mode=lax.GatherScatterMode.PROMISE_IN_BOUNDS)
    for j in range(LG):
        acc[pl.ds(j * L, L)] = acc[pl.ds(j * L, L)] / my_len

    pltpu.sync_copy(acc, spmem.at[s])
    plsc.subcore_barrier()

    @pl.when(h == 0)
    def _():
        pltpu.sync_copy(spmem.at[s + 1], buf0.at[0])
        for j in range(LG):
            acc[pl.ds(j * L, L)] = acc[pl.ds(j * L, L)] + buf0[0, pl.ds(j * L, L)]
        pltpu.sync_copy(acc, out_hbm.at[b])


TR = 1312


def _tc_body(lens_sm, x_ref, o_ref):
    t = pl.program_id(1)

    @pl.when(t == 0)
    def _():
        o_ref[...] = jnp.zeros_like(o_ref)

    o_ref[...] += jnp.sum(x_ref[...], axis=1, keepdims=True)

    @pl.when(t == pl.num_programs(1) - 1)
    def _():
        b = pl.program_id(0)
        lf = lens_sm[b].astype(jnp.float32)
        lf = jnp.where(lf == 0.0, jnp.float32(1.5), lf)
        o_ref[...] = o_ref[...] / lf


def _tc_mean(x, lens):
    return pl.pallas_call(
        _tc_body,
        grid_spec=pltpu.PrefetchScalarGridSpec(
            num_scalar_prefetch=1,
            grid=(B, TTC // TR),
            in_specs=[pl.BlockSpec(
                (1, TR, D), lambda b, t, lens_s: (b, t, 0))],
            out_specs=pl.BlockSpec((1, 1, D), lambda b, t, lens_s: (b, 0, 0)),
        ),
        out_shape=jax.ShapeDtypeStruct((B, 1, D), jnp.float32),
        compiler_params=pltpu.CompilerParams(
            dimension_semantics=("parallel", "arbitrary")),
    )(lens, x).reshape(B, D)


def kernel(x, lens):
    return _sc_mean(x, lens) + _tc_mean(x, lens)


def _sc_mean(x, lens):
    mesh = plsc.VectorSubcoreMesh(core_axis_name="c", subcore_axis_name="s")
    return pl.kernel(
        _body,
        out_type=jax.ShapeDtypeStruct((B, D), jnp.float32),
        mesh=mesh,
        scratch_types=[
            pltpu.VMEM((TCH, D), jnp.float32),
            pltpu.VMEM((TCH, D), jnp.float32),
            pltpu.VMEM((D,), jnp.float32),
            pltpu.VMEM((L,), jnp.int32),
            pltpu.VMEM_SHARED((NS, D), jnp.float32),
            pltpu.SemaphoreType.DMA,
            pltpu.SemaphoreType.DMA,
        ],
    )(x, lens)


# speedup vs baseline: 1.9690x; 1.0023x over previous
import jax
import jax.numpy as jnp
from jax import lax
from jax.experimental import pallas as pl
from jax.experimental.pallas import tpu as pltpu
from jax.experimental.pallas import tpu_sc as plsc

B, T, D = 16, 4096, 1024
NC, NS, L = 2, 16, 16
LG = D // L
TCH = 16
TTC = 2624
ROWS_W = (T - TTC) // 2
NCH = ROWS_W // TCH


def _tree(buf, col):
    vs = [buf[t, pl.ds(col, L)] for t in range(TCH)]
    while len(vs) > 1:
        nxt = [vs[i] + vs[i + 1] for i in range(0, len(vs) - 1, 2)]
        if len(vs) % 2:
            nxt.append(vs[-1])
        vs = nxt
    return vs[0]


def _body(x_hbm, lens_hbm, out_hbm, buf0, buf1, acc, lens_v, spmem, sem0, sem1):
    c = lax.axis_index("c")
    s = lax.axis_index("s")
    b = c * 8 + s // 2
    h = s % 2
    row0 = TTC + h * ROWS_W

    zeros = jnp.zeros((L,), jnp.float32)

    @plsc.parallel_loop(0, LG, 1, unroll=4)
    def _(j):
        acc[pl.ds(j * L, L)] = zeros

    def src(chunk):
        return x_hbm.at[b, pl.ds(row0 + chunk * TCH, TCH), :]

    pltpu.make_async_copy(src(0), buf0, sem0).start()
    pltpu.make_async_copy(src(1), buf1, sem1).start()

    def accumulate(buf):
        @plsc.parallel_loop(0, LG, 1, unroll=4)
        def _(j):
            col = j * L
            acc[pl.ds(col, L)] = acc[pl.ds(col, L)] + _tree(buf, col)

    def pair(i, _):
        c0 = 2 * i
        pltpu.make_async_copy(src(c0), buf0, sem0).wait()
        accumulate(buf0)

        @pl.when(c0 + 2 < NCH)
        def _():
            pltpu.make_async_copy(src(c0 + 2), buf0, sem0).start()

        pltpu.make_async_copy(src(c0 + 1), buf1, sem1).wait()
        accumulate(buf1)

        @pl.when(c0 + 3 < NCH)
        def _():
            pltpu.make_async_copy(src(c0 + 3), buf1, sem1).start()

        return 0

    lax.fori_loop(0, NCH // 2, pair, 0)

    pltpu.sync_copy(lens_hbm, lens_v)
    lens_f = lens_v[...].astype(jnp.float32)
    lens_f = jnp.where(lens_f == 0.0, jnp.float32(1.5), lens_f)
    idx = jnp.full((L,), b, dtype=jnp.int32)
    dnums = lax.GatherDimensionNumbers(
        offset_dims=(), collapsed_slice_dims=(0,), start_index_map=(0,))
    my_len = lax.gather(lens_f, idx[:, None], dnums, slice_sizes=(1,),
                        mode=lax.GatherScatterMode.PROMISE_IN_BOUNDS)
    @plsc.parallel_loop(0, LG, 1, unroll=4)
    def _(j):
        acc[pl.ds(j * L, L)] = acc[pl.ds(j * L, L)] / my_len

    pltpu.sync_copy(acc, spmem.at[s])
    plsc.subcore_barrier()

    @pl.when(h == 0)
    def _():
        pltpu.sync_copy(spmem.at[s + 1], buf0.at[0])

        @plsc.parallel_loop(0, LG, 1, unroll=4)
        def _(j):
            acc[pl.ds(j * L, L)] = acc[pl.ds(j * L, L)] + buf0[0, pl.ds(j * L, L)]

        pltpu.sync_copy(acc, out_hbm.at[b])


TR = 1312


def _tc_body(lens_sm, x_ref, o_ref):
    t = pl.program_id(1)

    @pl.when(t == 0)
    def _():
        o_ref[...] = jnp.zeros_like(o_ref)

    o_ref[...] += jnp.sum(x_ref[...], axis=1, keepdims=True)

    @pl.when(t == pl.num_programs(1) - 1)
    def _():
        b = pl.program_id(0)
        lf = lens_sm[b].astype(jnp.float32)
        lf = jnp.where(lf == 0.0, jnp.float32(1.5), lf)
        o_ref[...] = o_ref[...] / lf


def _tc_mean(x, lens):
    return pl.pallas_call(
        _tc_body,
        grid_spec=pltpu.PrefetchScalarGridSpec(
            num_scalar_prefetch=1,
            grid=(B, TTC // TR),
            in_specs=[pl.BlockSpec(
                (1, TR, D), lambda b, t, lens_s: (b, t, 0))],
            out_specs=pl.BlockSpec((1, 1, D), lambda b, t, lens_s: (b, 0, 0)),
        ),
        out_shape=jax.ShapeDtypeStruct((B, 1, D), jnp.float32),
        compiler_params=pltpu.CompilerParams(
            dimension_semantics=("parallel", "arbitrary")),
    )(lens, x).reshape(B, D)


def kernel(x, lens):
    return _sc_mean(x, lens) + _tc_mean(x, lens)


def _sc_mean(x, lens):
    mesh = plsc.VectorSubcoreMesh(core_axis_name="c", subcore_axis_name="s")
    return pl.kernel(
        _body,
        out_type=jax.ShapeDtypeStruct((B, D), jnp.float32),
        mesh=mesh,
        scratch_types=[
            pltpu.VMEM((TCH, D), jnp.float32),
            pltpu.VMEM((TCH, D), jnp.float32),
            pltpu.VMEM((D,), jnp.float32),
            pltpu.VMEM((L,), jnp.int32),
            pltpu.VMEM_SHARED((NS, D), jnp.float32),
            pltpu.SemaphoreType.DMA,
            pltpu.SemaphoreType.DMA,
        ],
    )(x, lens)


# speedup vs baseline: 1.9784x; 1.0048x over previous
import jax
import jax.numpy as jnp
from jax import lax
from jax.experimental import pallas as pl
from jax.experimental.pallas import tpu as pltpu
from jax.experimental.pallas import tpu_sc as plsc

B, T, D = 16, 4096, 1024
NC, NS, L = 2, 16, 16
LG = D // L
TCH = 16
TTC = 3328
ROWS_W = (T - TTC) // 2
NCH = ROWS_W // TCH


def _tree(buf, col):
    vs = [buf[t, pl.ds(col, L)] for t in range(TCH)]
    while len(vs) > 1:
        nxt = [vs[i] + vs[i + 1] for i in range(0, len(vs) - 1, 2)]
        if len(vs) % 2:
            nxt.append(vs[-1])
        vs = nxt
    return vs[0]


def _body(x_hbm, lens_hbm, out_hbm, buf0, buf1, acc, lens_v, spmem, sem0, sem1):
    c = lax.axis_index("c")
    s = lax.axis_index("s")
    b = c * 8 + s // 2
    h = s % 2
    row0 = TTC + h * ROWS_W

    zeros = jnp.zeros((L,), jnp.float32)

    @plsc.parallel_loop(0, LG, 1, unroll=4)
    def _(j):
        acc[pl.ds(j * L, L)] = zeros

    def src(chunk):
        return x_hbm.at[b, pl.ds(row0 + chunk * TCH, TCH), :]

    pltpu.make_async_copy(src(0), buf0, sem0).start()
    pltpu.make_async_copy(src(1), buf1, sem1).start()

    def accumulate(buf):
        @plsc.parallel_loop(0, LG, 1, unroll=4)
        def _(j):
            col = j * L
            acc[pl.ds(col, L)] = acc[pl.ds(col, L)] + _tree(buf, col)

    def pair(i, _):
        c0 = 2 * i
        pltpu.make_async_copy(src(c0), buf0, sem0).wait()
        accumulate(buf0)

        @pl.when(c0 + 2 < NCH)
        def _():
            pltpu.make_async_copy(src(c0 + 2), buf0, sem0).start()

        pltpu.make_async_copy(src(c0 + 1), buf1, sem1).wait()
        accumulate(buf1)

        @pl.when(c0 + 3 < NCH)
        def _():
            pltpu.make_async_copy(src(c0 + 3), buf1, sem1).start()

        return 0

    lax.fori_loop(0, NCH // 2, pair, 0)

    pltpu.sync_copy(lens_hbm, lens_v)
    lens_f = lens_v[...].astype(jnp.float32)
    lens_f = jnp.where(lens_f == 0.0, jnp.float32(1.5), lens_f)
    idx = jnp.full((L,), b, dtype=jnp.int32)
    dnums = lax.GatherDimensionNumbers(
        offset_dims=(), collapsed_slice_dims=(0,), start_index_map=(0,))
    my_len = lax.gather(lens_f, idx[:, None], dnums, slice_sizes=(1,),
                        mode=lax.GatherScatterMode.PROMISE_IN_BOUNDS)
    @plsc.parallel_loop(0, LG, 1, unroll=4)
    def _(j):
        acc[pl.ds(j * L, L)] = acc[pl.ds(j * L, L)] / my_len

    pltpu.sync_copy(acc, spmem.at[s])
    plsc.subcore_barrier()

    @pl.when(h == 0)
    def _():
        pltpu.sync_copy(spmem.at[s + 1], buf0.at[0])

        @plsc.parallel_loop(0, LG, 1, unroll=4)
        def _(j):
            acc[pl.ds(j * L, L)] = acc[pl.ds(j * L, L)] + buf0[0, pl.ds(j * L, L)]

        pltpu.sync_copy(acc, out_hbm.at[b])


TR = 1664


def _tc_body(lens_sm, x_ref, o_ref):
    t = pl.program_id(1)

    @pl.when(t == 0)
    def _():
        o_ref[...] = jnp.zeros_like(o_ref)

    o_ref[...] += jnp.sum(x_ref[...], axis=1, keepdims=True)

    @pl.when(t == pl.num_programs(1) - 1)
    def _():
        b = pl.program_id(0)
        lf = lens_sm[b].astype(jnp.float32)
        lf = jnp.where(lf == 0.0, jnp.float32(1.5), lf)
        o_ref[...] = o_ref[...] / lf


def _tc_mean(x, lens):
    return pl.pallas_call(
        _tc_body,
        grid_spec=pltpu.PrefetchScalarGridSpec(
            num_scalar_prefetch=1,
            grid=(B, TTC // TR),
            in_specs=[pl.BlockSpec(
                (1, TR, D), lambda b, t, lens_s: (b, t, 0))],
            out_specs=pl.BlockSpec((1, 1, D), lambda b, t, lens_s: (b, 0, 0)),
        ),
        out_shape=jax.ShapeDtypeStruct((B, 1, D), jnp.float32),
        compiler_params=pltpu.CompilerParams(
            dimension_semantics=("parallel", "arbitrary")),
    )(lens, x).reshape(B, D)


def kernel(x, lens):
    return _sc_mean(x, lens) + _tc_mean(x, lens)


def _sc_mean(x, lens):
    mesh = plsc.VectorSubcoreMesh(core_axis_name="c", subcore_axis_name="s")
    return pl.kernel(
        _body,
        out_type=jax.ShapeDtypeStruct((B, D), jnp.float32),
        mesh=mesh,
        scratch_types=[
            pltpu.VMEM((TCH, D), jnp.float32),
            pltpu.VMEM((TCH, D), jnp.float32),
            pltpu.VMEM((D,), jnp.float32),
            pltpu.VMEM((L,), jnp.int32),
            pltpu.VMEM_SHARED((NS, D), jnp.float32),
            pltpu.SemaphoreType.DMA,
            pltpu.SemaphoreType.DMA,
        ],
    )(x, lens)
